# Initial kernel scaffold; baseline (speedup 1.0000x reference)
import functools

import jax
import jax.numpy as jnp
from jax import lax
from jax.experimental import pallas as pl
from jax.experimental.pallas import tpu as pltpu
from jax.experimental.pallas import tpu_sc as plsc

NUM_FEATS = 26
SLOTS = 4
COLS = NUM_FEATS * SLOTS
COLS_PAD = 112
VOCAB = 1000
TAB_ROWS = NUM_FEATS * (VOCAB + 1)
EMB = 64
B = 4096
IN_DIM = NUM_FEATS * EMB
OUT_DIM = 4096
NUM_TOKENS = 8
D_MODEL = 512

NW = 32
ROWS_PER_W = B // NW
CB = 8
NCHUNK = ROWS_PER_W // CB

L = 16


def _sc_gather_body(ints_hbm, table_hbm, out_hbm, ints_v, idx_v, rows_v, out_v, sem):
    cid = lax.axis_index("c")
    sid = lax.axis_index("s")
    wid = sid * 2 + cid
    lane = lax.iota(jnp.int32, L)

    def chunk_body(ch, carry):
        base = wid * ROWS_PER_W + ch * CB
        pltpu.sync_copy(ints_hbm.at[pl.ds(base * COLS, CB * COLS)], ints_v)

        for r in range(CB):
            for t in range(COLS_PAD // L):
                j = jnp.minimum(t * L + lane, COLS - 1)
                vals = plsc.load_gather(ints_v, [r * COLS + j])
                vals = jnp.clip(vals, 0, VOCAB)
                gidx = vals + (j >> 2) * (VOCAB + 1)
                idx_v[r, pl.ds(t * L, L)] = gidx

        copies = [
            pltpu.async_copy(table_hbm.at[idx_v.at[r]], rows_v.at[r], sem)
            for r in range(CB)
        ]
        for c in copies:
            c.wait()

        def red_body(i, rcarry):
            b = i // NUM_FEATS
            f = i % NUM_FEATS
            for c in range(EMB // L):
                sl = pl.ds(c * L, L)
                acc = (rows_v[b, 4 * f, sl] + rows_v[b, 4 * f + 1, sl]
                       + rows_v[b, 4 * f + 2, sl] + rows_v[b, 4 * f + 3, sl])
                out_v[b, pl.ds(f * EMB + c * L, L)] = acc * 0.25
            return rcarry

        lax.fori_loop(0, CB * NUM_FEATS, red_body, 0)
        pltpu.sync_copy(out_v, out_hbm.at[pl.ds(base, CB)])
        return carry

    lax.fori_loop(0, NCHUNK, chunk_body, 0)


@jax.jit
def _sc_gather(ints_flat, table_flat):
    mesh = plsc.VectorSubcoreMesh(core_axis_name="c", subcore_axis_name="s")
    return pl.kernel(
        _sc_gather_body,
        out_type=jax.ShapeDtypeStruct((B, IN_DIM), jnp.float32),
        mesh=mesh,
        scratch_types=[
            pltpu.VMEM((CB * COLS,), jnp.int32),
            pltpu.VMEM((CB, COLS_PAD), jnp.int32),
            pltpu.VMEM((CB, COLS_PAD, EMB), jnp.float32),
            pltpu.VMEM((CB, IN_DIM), jnp.float32),
            pltpu.SemaphoreType.DMA,
        ],
    )(ints_flat, table_flat)


BT = 256
NT = 512
N_STEPS = OUT_DIM // NT


def _dense_body(x_ref, w_ref, b_ref, g_ref, be_ref, o_ref, acc_ref):
    n = pl.program_id(1)
    h = jnp.dot(x_ref[...], w_ref[...], preferred_element_type=jnp.float32)
    h = h + b_ref[...]
    h = h * jax.nn.sigmoid(h)
    acc_ref[:, pl.ds(n * NT, NT)] = h

    @pl.when(n == N_STEPS - 1)
    def _():
        a = acc_ref[...]
        mu = jnp.mean(a, axis=1, keepdims=True)
        d = a - mu
        var = jnp.mean(d * d, axis=1, keepdims=True)
        o_ref[...] = d * lax.rsqrt(var + 1e-5) * g_ref[...] + be_ref[...]


@jax.jit
def _tc_dense(ft, W1, b1, g, be):
    return pl.pallas_call(
        _dense_body,
        grid=(B // BT, N_STEPS),
        in_specs=[
            pl.BlockSpec((BT, IN_DIM), lambda b, n: (b, 0)),
            pl.BlockSpec((IN_DIM, NT), lambda b, n: (0, n)),
            pl.BlockSpec((1, NT), lambda b, n: (0, n)),
            pl.BlockSpec((1, OUT_DIM), lambda b, n: (0, 0)),
            pl.BlockSpec((1, OUT_DIM), lambda b, n: (0, 0)),
        ],
        out_specs=pl.BlockSpec((BT, OUT_DIM), lambda b, n: (b, 0)),
        out_shape=jax.ShapeDtypeStruct((B, OUT_DIM), jnp.float32),
        scratch_shapes=[pltpu.VMEM((BT, OUT_DIM), jnp.float32)],
        compiler_params=pltpu.CompilerParams(
            dimension_semantics=("parallel", "arbitrary"),
        ),
    )(ft, W1, b1, g, be)


def kernel(int_feats, emb_tables, W1, b1, ln_gamma, ln_beta):
    ft = _sc_gather(int_feats.reshape(-1).astype(jnp.int32),
                    emb_tables.reshape(TAB_ROWS, EMB))
    h = _tc_dense(ft, W1, b1.reshape(1, OUT_DIM),
                  ln_gamma.reshape(1, OUT_DIM), ln_beta.reshape(1, OUT_DIM))
    return h.reshape(B, NUM_TOKENS, D_MODEL)


# speedup vs baseline: 4.7282x; 4.7282x over previous
import functools

import jax
import jax.numpy as jnp
from jax import lax
from jax.experimental import pallas as pl
from jax.experimental.pallas import tpu as pltpu
from jax.experimental.pallas import tpu_sc as plsc

NUM_FEATS = 26
SLOTS = 4
COLS = NUM_FEATS * SLOTS
COLS_PAD = 112
VOCAB = 1000
TAB_ROWS = NUM_FEATS * (VOCAB + 1)
EMB = 64
B = 4096
IN_DIM = NUM_FEATS * EMB
OUT_DIM = 4096
NUM_TOKENS = 8
D_MODEL = 512

NW = 32
ROWS_PER_W = B // NW
CB = 8
NCHUNK = ROWS_PER_W // CB

L = 16


def _sc_gather_body(ints_hbm, table_hbm, out_hbm, ints_v, idx_v, rows_v, out_v, sem):
    cid = lax.axis_index("c")
    sid = lax.axis_index("s")
    wid = sid * 2 + cid
    lane = lax.iota(jnp.int32, L)

    def chunk_body(ch, carry):
        base = wid * ROWS_PER_W + ch * CB
        pltpu.sync_copy(ints_hbm.at[pl.ds(base * COLS, CB * COLS)], ints_v)

        for t in range(CB * COLS // L):
            p = t * L + lane
            j = lax.rem(p, jnp.full((L,), COLS, jnp.int32))
            vals = jnp.clip(ints_v[pl.ds(t * L, L)], 0, VOCAB)
            idx_v[pl.ds(t * L, L)] = vals + (j >> 2) * (VOCAB + 1)

        copies = [
            pltpu.async_copy(table_hbm.at[idx_v.at[pl.ds(r * COLS, COLS)]],
                             rows_v.at[r], sem)
            for r in range(CB)
        ]
        for c in copies:
            c.wait()

        def red_body(i, rcarry):
            b = i // NUM_FEATS
            f = i % NUM_FEATS
            for c in range(EMB // L):
                sl = pl.ds(c * L, L)
                acc = (rows_v[b, 4 * f, sl] + rows_v[b, 4 * f + 1, sl]
                       + rows_v[b, 4 * f + 2, sl] + rows_v[b, 4 * f + 3, sl])
                out_v[b, pl.ds(f * EMB + c * L, L)] = acc * 0.25
            return rcarry

        lax.fori_loop(0, CB * NUM_FEATS, red_body, 0)
        pltpu.sync_copy(out_v, out_hbm.at[pl.ds(base, CB)])
        return carry

    lax.fori_loop(0, NCHUNK, chunk_body, 0)


@jax.jit
def _sc_gather(ints_flat, table_flat):
    mesh = plsc.VectorSubcoreMesh(core_axis_name="c", subcore_axis_name="s")
    return pl.kernel(
        _sc_gather_body,
        out_type=jax.ShapeDtypeStruct((B, IN_DIM), jnp.float32),
        mesh=mesh,
        scratch_types=[
            pltpu.VMEM((CB * COLS,), jnp.int32),
            pltpu.VMEM((CB * COLS,), jnp.int32),
            pltpu.VMEM((CB, COLS, EMB), jnp.float32),
            pltpu.VMEM((CB, IN_DIM), jnp.float32),
            pltpu.SemaphoreType.DMA,
        ],
        compiler_params=pltpu.CompilerParams(use_tc_tiling_on_sc=False),
    )(ints_flat, table_flat)


BT = 256
NT = 512
N_STEPS = OUT_DIM // NT


def _dense_body(x_ref, w_ref, b_ref, g_ref, be_ref, o_ref, acc_ref):
    n = pl.program_id(1)
    h = jnp.dot(x_ref[...], w_ref[...], preferred_element_type=jnp.float32)
    h = h + b_ref[...]
    h = h * jax.nn.sigmoid(h)
    acc_ref[:, pl.ds(n * NT, NT)] = h

    @pl.when(n == N_STEPS - 1)
    def _():
        a = acc_ref[...]
        mu = jnp.mean(a, axis=1, keepdims=True)
        d = a - mu
        var = jnp.mean(d * d, axis=1, keepdims=True)
        o_ref[...] = d * lax.rsqrt(var + 1e-5) * g_ref[...] + be_ref[...]


@jax.jit
def _tc_dense(ft, W1, b1, g, be):
    return pl.pallas_call(
        _dense_body,
        grid=(B // BT, N_STEPS),
        in_specs=[
            pl.BlockSpec((BT, IN_DIM), lambda b, n: (b, 0)),
            pl.BlockSpec((IN_DIM, NT), lambda b, n: (0, n)),
            pl.BlockSpec((1, NT), lambda b, n: (0, n)),
            pl.BlockSpec((1, OUT_DIM), lambda b, n: (0, 0)),
            pl.BlockSpec((1, OUT_DIM), lambda b, n: (0, 0)),
        ],
        out_specs=pl.BlockSpec((BT, OUT_DIM), lambda b, n: (b, 0)),
        out_shape=jax.ShapeDtypeStruct((B, OUT_DIM), jnp.float32),
        scratch_shapes=[pltpu.VMEM((BT, OUT_DIM), jnp.float32)],
        compiler_params=pltpu.CompilerParams(
            dimension_semantics=("parallel", "arbitrary"),
        ),
    )(ft, W1, b1, g, be)


def kernel(int_feats, emb_tables, W1, b1, ln_gamma, ln_beta):
    ft = _sc_gather(int_feats.reshape(-1).astype(jnp.int32),
                    emb_tables.reshape(TAB_ROWS, EMB))
    h = _tc_dense(ft, W1, b1.reshape(1, OUT_DIM),
                  ln_gamma.reshape(1, OUT_DIM), ln_beta.reshape(1, OUT_DIM))
    return h.reshape(B, NUM_TOKENS, D_MODEL)


# speedup vs baseline: 5.7098x; 1.2076x over previous
import functools

import jax
import jax.numpy as jnp
import numpy as np
from jax import lax
from jax.experimental import pallas as pl
from jax.experimental.pallas import tpu as pltpu
from jax.experimental.pallas import tpu_sc as plsc

NUM_FEATS = 26
SLOTS = 4
COLS = NUM_FEATS * SLOTS
COLS_PAD = 112
VOCAB = 1000
TAB_ROWS = NUM_FEATS * (VOCAB + 1)
EMB = 64
B = 4096
IN_DIM = NUM_FEATS * EMB
OUT_DIM = 4096
NUM_TOKENS = 8
D_MODEL = 512

NW = 32
ROWS_PER_W = B // NW
CB = 8
NCHUNK = ROWS_PER_W // CB

L = 16


def _sc_gather_body(ints_hbm, table_hbm, out_hbm, ints_v, idx_v, off_v, rows_v,
                    out_v, sem):
    cid = lax.axis_index("c")
    sid = lax.axis_index("s")
    wid = sid * 2 + cid
    lane = lax.iota(jnp.int32, L)
    colsv = jnp.full((L,), COLS, jnp.int32)

    for t in range(CB * COLS // L):
        j = lax.rem(t * L + lane, colsv)
        off_v[pl.ds(t * L, L)] = (j >> 2) * (VOCAB + 1)

    def chunk_body(ch, carry):
        base = wid * ROWS_PER_W + ch * CB
        pltpu.sync_copy(ints_hbm.at[pl.ds(base * COLS, CB * COLS)], ints_v)

        for t in range(CB * COLS // L):
            sl = pl.ds(t * L, L)
            idx_v[sl] = jnp.clip(ints_v[sl], 0, VOCAB) + off_v[sl]

        copies = [
            pltpu.async_copy(table_hbm.at[idx_v.at[pl.ds(r * COLS, COLS)]],
                             rows_v.at[r], sem)
            for r in range(CB)
        ]
        for c in copies:
            c.wait()

        def red_b(b, bcarry):
            def red_f(f, fcarry):
                for c in range(EMB // L):
                    sl = pl.ds(c * L, L)
                    acc = (rows_v[b, 4 * f, sl] + rows_v[b, 4 * f + 1, sl]
                           + rows_v[b, 4 * f + 2, sl] + rows_v[b, 4 * f + 3, sl])
                    out_v[b, pl.ds(f * EMB + c * L, L)] = acc * 0.25
                return fcarry
            return lax.fori_loop(0, NUM_FEATS, red_f, bcarry)

        lax.fori_loop(0, CB, red_b, 0)
        pltpu.sync_copy(out_v, out_hbm.at[pl.ds(base, CB)])
        return carry

    lax.fori_loop(0, NCHUNK, chunk_body, 0)


@jax.jit
def _sc_gather(ints_flat, table_flat):
    mesh = plsc.VectorSubcoreMesh(core_axis_name="c", subcore_axis_name="s")
    return pl.kernel(
        _sc_gather_body,
        out_type=jax.ShapeDtypeStruct((B, IN_DIM), jnp.float32),
        mesh=mesh,
        scratch_types=[
            pltpu.VMEM((CB * COLS,), jnp.int32),
            pltpu.VMEM((CB * COLS,), jnp.int32),
            pltpu.VMEM((CB * COLS,), jnp.int32),
            pltpu.VMEM((CB, COLS, EMB), jnp.float32),
            pltpu.VMEM((CB, IN_DIM), jnp.float32),
            pltpu.SemaphoreType.DMA,
        ],
        compiler_params=pltpu.CompilerParams(use_tc_tiling_on_sc=False),
    )(ints_flat, table_flat)


BT = 512
NT = 512
N_STEPS = OUT_DIM // NT


def _dense_body(x_ref, w_ref, b_ref, g_ref, be_ref, o_ref, acc_ref):
    n = pl.program_id(1)
    h = jnp.dot(x_ref[...].astype(jnp.bfloat16), w_ref[...],
                preferred_element_type=jnp.float32)
    h = h + b_ref[...]
    h = h * jax.nn.sigmoid(h)
    acc_ref[:, pl.ds(n * NT, NT)] = h

    @pl.when(n == N_STEPS - 1)
    def _():
        a = acc_ref[...]
        mu = jnp.mean(a, axis=1, keepdims=True)
        d = a - mu
        var = jnp.mean(d * d, axis=1, keepdims=True)
        o_ref[...] = d * lax.rsqrt(var + 1e-5) * g_ref[...] + be_ref[...]


@jax.jit
def _tc_dense(ft, W1, b1, g, be):
    return pl.pallas_call(
        _dense_body,
        grid=(B // BT, N_STEPS),
        in_specs=[
            pl.BlockSpec((BT, IN_DIM), lambda b, n: (b, 0)),
            pl.BlockSpec((IN_DIM, NT), lambda b, n: (0, n)),
            pl.BlockSpec((1, NT), lambda b, n: (0, n)),
            pl.BlockSpec((1, OUT_DIM), lambda b, n: (0, 0)),
            pl.BlockSpec((1, OUT_DIM), lambda b, n: (0, 0)),
        ],
        out_specs=pl.BlockSpec((BT, OUT_DIM), lambda b, n: (b, 0)),
        out_shape=jax.ShapeDtypeStruct((B, OUT_DIM), jnp.float32),
        scratch_shapes=[pltpu.VMEM((BT, OUT_DIM), jnp.float32)],
        compiler_params=pltpu.CompilerParams(
            dimension_semantics=("parallel", "arbitrary"),
        ),
    )(ft, W1, b1, g, be)


def kernel(int_feats, emb_tables, W1, b1, ln_gamma, ln_beta):
    ft = _sc_gather(int_feats.reshape(-1).astype(jnp.int32),
                    emb_tables.reshape(TAB_ROWS, EMB))
    h = _tc_dense(ft, W1.astype(jnp.bfloat16), b1.reshape(1, OUT_DIM),
                  ln_gamma.reshape(1, OUT_DIM), ln_beta.reshape(1, OUT_DIM))
    return h.reshape(B, NUM_TOKENS, D_MODEL)


# speedup vs baseline: 6.2533x; 1.0952x over previous
import functools

import jax
import jax.numpy as jnp
import numpy as np
from jax import lax
from jax.experimental import pallas as pl
from jax.experimental.pallas import tpu as pltpu
from jax.experimental.pallas import tpu_sc as plsc

NUM_FEATS = 26
SLOTS = 4
COLS = NUM_FEATS * SLOTS
COLS_PAD = 112
VOCAB = 1000
TAB_ROWS = NUM_FEATS * (VOCAB + 1)
EMB = 64
B = 4096
IN_DIM = NUM_FEATS * EMB
OUT_DIM = 4096
NUM_TOKENS = 8
D_MODEL = 512

NW = 32
ROWS_PER_W = B // NW
CB = 8
NCHUNK = ROWS_PER_W // CB

L = 16


def _sc_gather_body(ints_hbm, table_hbm, out_hbm, ints_v, idx_v, off_v, rows_v,
                    out_v, sems):
    cid = lax.axis_index("c")
    sid = lax.axis_index("s")
    wid = sid * 2 + cid
    lane = lax.iota(jnp.int32, L)
    colsv = jnp.full((L,), COLS, jnp.int32)

    for t in range(CB * COLS // L):
        j = lax.rem(t * L + lane, colsv)
        off_v[pl.ds(t * L, L)] = (j >> 2) * (VOCAB + 1)

    def fire(ch, buf):
        base = wid * ROWS_PER_W + ch * CB
        pltpu.sync_copy(ints_hbm.at[pl.ds(base * COLS, CB * COLS)],
                        ints_v.at[buf])
        for t in range(CB * COLS // L):
            sl = pl.ds(t * L, L)
            idx_v[buf, sl] = jnp.clip(ints_v[buf, sl], 0, VOCAB) + off_v[sl]
        for r in range(CB):
            pltpu.async_copy(
                table_hbm.at[idx_v.at[buf, pl.ds(r * COLS, COLS)]],
                rows_v.at[buf, r], sems.at[buf])

    def drain_reduce(ch, buf):
        base = wid * ROWS_PER_W + ch * CB
        for r in range(CB):
            pltpu.make_async_copy(
                table_hbm.at[idx_v.at[buf, pl.ds(r * COLS, COLS)]],
                rows_v.at[buf, r], sems.at[buf]).wait()

        def red_b(b, bcarry):
            def red_f(f, fcarry):
                for c in range(EMB // L):
                    sl = pl.ds(c * L, L)
                    acc = (rows_v[buf, b, 4 * f, sl]
                           + rows_v[buf, b, 4 * f + 1, sl]
                           + rows_v[buf, b, 4 * f + 2, sl]
                           + rows_v[buf, b, 4 * f + 3, sl])
                    out_v[b, pl.ds(f * EMB + c * L, L)] = acc * 0.25
                return fcarry
            return lax.fori_loop(0, NUM_FEATS, red_f, bcarry)

        lax.fori_loop(0, CB, red_b, 0)
        pltpu.sync_copy(out_v, out_hbm.at[pl.ds(base, CB)])

    fire(0, 0)

    def chunk_body(ch, carry):
        buf = lax.rem(ch, 2)

        @pl.when(ch + 1 < NCHUNK)
        def _():
            fire(ch + 1, 1 - buf)

        drain_reduce(ch, buf)
        return carry

    lax.fori_loop(0, NCHUNK, chunk_body, 0)


@jax.jit
def _sc_gather(ints_flat, table_flat):
    mesh = plsc.VectorSubcoreMesh(core_axis_name="c", subcore_axis_name="s")
    return pl.kernel(
        _sc_gather_body,
        out_type=jax.ShapeDtypeStruct((B, IN_DIM), jnp.float32),
        mesh=mesh,
        scratch_types=[
            pltpu.VMEM((2, CB * COLS), jnp.int32),
            pltpu.VMEM((2, CB * COLS), jnp.int32),
            pltpu.VMEM((CB * COLS,), jnp.int32),
            pltpu.VMEM((2, CB, COLS, EMB), jnp.float32),
            pltpu.VMEM((CB, IN_DIM), jnp.float32),
            pltpu.SemaphoreType.DMA((2,)),
        ],
        compiler_params=pltpu.CompilerParams(use_tc_tiling_on_sc=False),
    )(ints_flat, table_flat)


BT = 512
NT = 512
N_STEPS = OUT_DIM // NT


def _dense_body(x_ref, w_ref, b_ref, g_ref, be_ref, o_ref, acc_ref):
    n = pl.program_id(1)
    h = jnp.dot(x_ref[...].astype(jnp.bfloat16), w_ref[...],
                preferred_element_type=jnp.float32)
    h = h + b_ref[...]
    h = h * jax.nn.sigmoid(h)
    acc_ref[:, pl.ds(n * NT, NT)] = h

    @pl.when(n == N_STEPS - 1)
    def _():
        a = acc_ref[...]
        mu = jnp.mean(a, axis=1, keepdims=True)
        d = a - mu
        var = jnp.mean(d * d, axis=1, keepdims=True)
        o_ref[...] = d * lax.rsqrt(var + 1e-5) * g_ref[...] + be_ref[...]


@jax.jit
def _tc_dense(ft, W1, b1, g, be):
    return pl.pallas_call(
        _dense_body,
        grid=(B // BT, N_STEPS),
        in_specs=[
            pl.BlockSpec((BT, IN_DIM), lambda b, n: (b, 0)),
            pl.BlockSpec((IN_DIM, NT), lambda b, n: (0, n)),
            pl.BlockSpec((1, NT), lambda b, n: (0, n)),
            pl.BlockSpec((1, OUT_DIM), lambda b, n: (0, 0)),
            pl.BlockSpec((1, OUT_DIM), lambda b, n: (0, 0)),
        ],
        out_specs=pl.BlockSpec((BT, OUT_DIM), lambda b, n: (b, 0)),
        out_shape=jax.ShapeDtypeStruct((B, OUT_DIM), jnp.float32),
        scratch_shapes=[pltpu.VMEM((BT, OUT_DIM), jnp.float32)],
        compiler_params=pltpu.CompilerParams(
            dimension_semantics=("parallel", "arbitrary"),
        ),
    )(ft, W1, b1, g, be)


def kernel(int_feats, emb_tables, W1, b1, ln_gamma, ln_beta):
    ft = _sc_gather(int_feats.reshape(-1).astype(jnp.int32),
                    emb_tables.reshape(TAB_ROWS, EMB))
    h = _tc_dense(ft, W1.astype(jnp.bfloat16), b1.reshape(1, OUT_DIM),
                  ln_gamma.reshape(1, OUT_DIM), ln_beta.reshape(1, OUT_DIM))
    return h.reshape(B, NUM_TOKENS, D_MODEL)
